# baseline (speedup 1.0000x reference)
import functools

import jax
import jax.numpy as jnp
from jax import lax
from jax.experimental import pallas as pl
from jax.experimental.pallas import tpu as pltpu
from jax.experimental.pallas import tpu_sc as plsc

_N = 10000
_D = 128
_E = 320000

_NC = 2
_NS = 16
_NW = _NC * _NS
_CH = 128
_NCHUNK = 78
_EPW = _E // _NW
_TAIL = _EPW - _NCHUNK * _CH
_ROWS = 10112
_RPS = _ROWS // _NS
_BLOCKS = ((0, 128), (128, 128), (256, 128), (384, 128), (512, 120))


def _segsum_body(h_hbm, src_hbm, dst_hbm, out_hbm, acc,
                 rows0, is0, id0, g0, s0,
                 rows1, is1, id1, g1, s1,
                 rows2, is2, id2, g2, s2,
                 ist, idt):
    c = lax.axis_index("c")
    s = lax.axis_index("s")
    wid = s * _NC + c

    def _zrow(r, _):
        def _zcol(k, __):
            rows0[r, pl.ds(k * 16, 16)] = jnp.zeros((16,), jnp.float32)
            return 0
        return lax.fori_loop(0, _D // 16, _zcol, 0)
    lax.fori_loop(0, _CH, _zrow, 0)
    for r0b, nb in _BLOCKS:
        pltpu.async_copy(rows0.at[pl.ds(0, nb)],
                         acc.at[pl.ds(s * _RPS + r0b, nb)], g0)
    for r0b, nb in _BLOCKS:
        pltpu.make_async_copy(rows0.at[pl.ds(0, nb)],
                              acc.at[pl.ds(s * _RPS + r0b, nb)], g0).wait()
    plsc.subcore_barrier()

    slots = ((rows0, is0, id0, g0, s0),
             (rows1, is1, id1, g1, s1),
             (rows2, is2, id2, g2, s2))
    e0 = wid * _EPW
    for k, (rw, isk, idk, gk, sk) in enumerate(slots):
        off = e0 + k * _CH
        pltpu.sync_copy(src_hbm.at[pl.ds(off, _CH)], isk)
        pltpu.sync_copy(dst_hbm.at[pl.ds(off, _CH)], idk)
        pltpu.async_copy(h_hbm.at[isk], rw, gk)

    def _iter(i, _):
        for rw, isk, idk, gk, sk in slots:
            pltpu.make_async_copy(h_hbm.at[isk], rw, gk).wait()
            pltpu.async_copy(rw, acc.at[idk], sk, add=True)
        for k, (rw, isk, idk, gk, sk) in enumerate(slots):
            jn = 3 * i + k + 3

            @pl.when(jn < _NCHUNK)
            def _(rw=rw, isk=isk, idk=idk, gk=gk, sk=sk, jn=jn):
                pltpu.make_async_copy(rw, acc.at[idk], sk).wait()
                off = e0 + jn * _CH
                pltpu.sync_copy(src_hbm.at[pl.ds(off, _CH)], isk)
                pltpu.sync_copy(dst_hbm.at[pl.ds(off, _CH)], idk)
                pltpu.async_copy(h_hbm.at[isk], rw, gk)
        return 0
    lax.fori_loop(0, _NCHUNK // 3, _iter, 0)
    for rw, isk, idk, gk, sk in slots:
        pltpu.make_async_copy(rw, acc.at[idk], sk).wait()

    ot = e0 + _NCHUNK * _CH
    pltpu.sync_copy(src_hbm.at[pl.ds(ot, _TAIL)], ist)
    pltpu.sync_copy(dst_hbm.at[pl.ds(ot, _TAIL)], idt)
    pltpu.async_copy(h_hbm.at[ist], rows0.at[pl.ds(0, _TAIL)], g0).wait()
    pltpu.sync_copy(rows0.at[pl.ds(0, _TAIL)], acc.at[idt], add=True)
    plsc.subcore_barrier()

    def _wb_in(idx):
        r0b, nb = _BLOCKS[idx]
        rw = slots[idx % 3][0]
        return (acc.at[pl.ds(s * _RPS + r0b, nb)], rw.at[pl.ds(0, nb)],
                slots[idx % 3][3])
    def _wb_out(idx):
        r0b, nb = _BLOCKS[idx]
        rw = slots[idx % 3][0]
        return (rw.at[pl.ds(0, nb)], out_hbm.at[c, pl.ds(s * _RPS + r0b, nb)],
                slots[idx % 3][4])
    for idx in range(len(_BLOCKS)):
        if idx >= 3:
            pltpu.make_async_copy(*_wb_out(idx - 3)).wait()
        pltpu.async_copy(*_wb_in(idx))
        pltpu.make_async_copy(*_wb_in(idx)).wait()
        pltpu.async_copy(*_wb_out(idx))
    for idx in range(max(0, len(_BLOCKS) - 3), len(_BLOCKS)):
        pltpu.make_async_copy(*_wb_out(idx)).wait()


def _segsum(h, src, dst):
    mesh = plsc.VectorSubcoreMesh(core_axis_name="c", subcore_axis_name="s")
    f = pl.kernel(
        _segsum_body,
        mesh=mesh,
        out_type=jax.ShapeDtypeStruct((_NC, _ROWS, _D), jnp.float32),
        scratch_types=[
            pltpu.VMEM_SHARED((_ROWS, _D), jnp.float32),
        ] + [
            t for _k in range(3) for t in (
                pltpu.VMEM((_CH, _D), jnp.float32),
                pltpu.VMEM((_CH,), jnp.int32),
                pltpu.VMEM((_CH,), jnp.int32),
                pltpu.SemaphoreType.DMA,
                pltpu.SemaphoreType.DMA,
            )
        ] + [
            pltpu.VMEM((_TAIL,), jnp.int32),
            pltpu.VMEM((_TAIL,), jnp.int32),
        ],
    )
    return f(h, src, dst)


def _matmul_t(x, w):
    return lax.dot_general(x, w, (((1,), (1,)), ((), ())),
                           preferred_element_type=jnp.float32)


def _lin2_body(xa_ref, wa_ref, ba_ref, xp_ref, wp_ref, bp_ref, oa_ref, op_ref):
    oa_ref[...] = _matmul_t(xa_ref[...], wa_ref[...]) + ba_ref[...]
    op_ref[...] = _matmul_t(xp_ref[...], wp_ref[...]) + bp_ref[...]


def _lin2(xa, pa, xp, pp):
    return pl.pallas_call(
        _lin2_body,
        out_shape=[jax.ShapeDtypeStruct((_N, _D), jnp.float32)] * 2,
    )(xa, pa["W"], pa["b"].reshape(1, -1), xp, pp["W"], pp["b"].reshape(1, -1))


def _mlp_core(x_ref, a_ref, eps_ref, w1_ref, b1_ref, g_ref, be_ref, w2_ref, b2_ref):
    agg = a_ref[0, 0:_N, :] + a_ref[1, 0:_N, :]
    h = (1.0 + eps_ref[0]) * x_ref[...] + agg
    t = _matmul_t(h, w1_ref[...]) + b1_ref[...]
    mean = jnp.mean(t, axis=0, keepdims=True)
    var = jnp.mean((t - mean) ** 2, axis=0, keepdims=True)
    t = (t - mean) * lax.rsqrt(var + 1e-5) * g_ref[...] + be_ref[...]
    t = jnp.maximum(t, 0.0)
    t = _matmul_t(t, w2_ref[...]) + b2_ref[...]
    return jnp.maximum(t, 0.0)


def _gin_mlp2_body(xw_ref, aw_ref, ew_ref, w1w_ref, b1w_ref, gw_ref, bew_ref,
                   w2w_ref, b2w_ref,
                   xn_ref, an_ref, en_ref, w1n_ref, b1n_ref, gn_ref, ben_ref,
                   w2n_ref, b2n_ref, ow_ref, on_ref):
    ow_ref[...] = _mlp_core(xw_ref, aw_ref, ew_ref, w1w_ref, b1w_ref, gw_ref,
                            bew_ref, w2w_ref, b2w_ref)
    on_ref[...] = _mlp_core(xn_ref, an_ref, en_ref, w1n_ref, b1n_ref, gn_ref,
                            ben_ref, w2n_ref, b2n_ref)


def _gin_mlp_final_body(x_ref, a_ref, eps_ref, w1_ref, b1_ref, g_ref, be_ref,
                        w2_ref, b2_ref, wf_ref, bf_ref, o_ref):
    t = _mlp_core(x_ref, a_ref, eps_ref, w1_ref, b1_ref, g_ref,
                  be_ref, w2_ref, b2_ref)
    o_ref[...] = _matmul_t(t, wf_ref[...]) + bf_ref[...]


def _mlp_args(x, agg, p):
    return (x, agg, p["eps"].reshape(1),
            p["W1"], p["b1"].reshape(1, -1),
            p["gamma"].reshape(1, -1), p["beta"].reshape(1, -1),
            p["W2"], p["b2"].reshape(1, -1))


_SMEM1 = pl.BlockSpec(memory_space=pltpu.SMEM)


def _gin_mlp2(xw, aggw, pw, xn, aggn, pn):
    specs = ([None, None, _SMEM1] + [None] * 6) * 2
    specs = [s if s is not None else pl.BlockSpec() for s in specs]
    return pl.pallas_call(
        _gin_mlp2_body,
        in_specs=specs,
        out_shape=[jax.ShapeDtypeStruct((_N, _D), jnp.float32)] * 2,
    )(*_mlp_args(xw, aggw, pw), *_mlp_args(xn, aggn, pn))


def _gin_mlp_final(x, agg, p, pf):
    specs = [None, None, _SMEM1] + [None] * 8
    specs = [s if s is not None else pl.BlockSpec() for s in specs]
    return pl.pallas_call(
        _gin_mlp_final_body,
        in_specs=specs,
        out_shape=jax.ShapeDtypeStruct((_N, pf["W"].shape[0]), jnp.float32),
    )(*_mlp_args(x, agg, p), pf["W"], pf["b"].reshape(1, -1))


def kernel(x_author, x_paper, params, ei_writes, ei_written):
    p = params
    src_w, dst_w = ei_writes[0], ei_writes[1]
    src_n, dst_n = ei_written[0], ei_written[1]

    h_a, h_p = _lin2(x_author, p["lin_author"], x_paper, p["lin_paper"])

    l1, l2 = p["layers"][0], p["layers"][1]
    agg_p = _segsum(h_a, src_w, dst_w)
    agg_a = _segsum(h_p, src_n, dst_n)
    h_p1, h_a1 = _gin_mlp2(h_p, agg_p, l1["writes"], h_a, agg_a, l1["written"])

    agg_a2 = _segsum(h_p1, src_n, dst_n)
    return _gin_mlp_final(h_a1, agg_a2, l2["written"], p["final"])


# speedup vs baseline: 1.0344x; 1.0344x over previous
import functools

import jax
import jax.numpy as jnp
from jax import lax
from jax.experimental import pallas as pl
from jax.experimental.pallas import tpu as pltpu
from jax.experimental.pallas import tpu_sc as plsc

_N = 10000
_D = 128
_E = 320000

_NC = 2
_NS = 16
_NW = _NC * _NS
_CH = 128
_NCHUNK = 78
_EPW = _E // _NW
_TAIL = _EPW - _NCHUNK * _CH
_ROWS = 10112
_RPS = _ROWS // _NS
_BLOCKS = ((0, 128), (128, 128), (256, 128), (384, 128), (512, 120))


def _segsum_body(h_hbm, src_hbm, dst_hbm, out_hbm, acc,
                 rows0, is0, id0, g0, s0,
                 rows1, is1, id1, g1, s1,
                 rows2, is2, id2, g2, s2,
                 ist, idt):
    c = lax.axis_index("c")
    s = lax.axis_index("s")
    wid = s * _NC + c

    def _zrow(r, _):
        def _zcol(k, __):
            rows0[r, pl.ds(k * 16, 16)] = jnp.zeros((16,), jnp.float32)
            return 0
        return lax.fori_loop(0, _D // 16, _zcol, 0)
    lax.fori_loop(0, _CH, _zrow, 0)
    for r0b, nb in _BLOCKS:
        pltpu.async_copy(rows0.at[pl.ds(0, nb)],
                         acc.at[pl.ds(s * _RPS + r0b, nb)], g0)
    for r0b, nb in _BLOCKS:
        pltpu.make_async_copy(rows0.at[pl.ds(0, nb)],
                              acc.at[pl.ds(s * _RPS + r0b, nb)], g0).wait()
    plsc.subcore_barrier()

    slots = ((rows0, is0, id0, g0, s0),
             (rows1, is1, id1, g1, s1),
             (rows2, is2, id2, g2, s2))
    e0 = wid * _EPW
    for k, (rw, isk, idk, gk, sk) in enumerate(slots):
        off = e0 + k * _CH
        pltpu.sync_copy(src_hbm.at[pl.ds(off, _CH)], isk)
        pltpu.sync_copy(dst_hbm.at[pl.ds(off, _CH)], idk)
        pltpu.async_copy(h_hbm.at[isk], rw, gk)

    def _iter(i, _):
        for rw, isk, idk, gk, sk in slots:
            pltpu.make_async_copy(h_hbm.at[isk], rw, gk).wait()
            pltpu.async_copy(rw, acc.at[idk], sk, add=True)
        for k, (rw, isk, idk, gk, sk) in enumerate(slots):
            jn = 3 * i + k + 3

            @pl.when(jn < _NCHUNK)
            def _(rw=rw, isk=isk, idk=idk, gk=gk, sk=sk, jn=jn):
                pltpu.make_async_copy(rw, acc.at[idk], sk).wait()
                off = e0 + jn * _CH
                pltpu.sync_copy(src_hbm.at[pl.ds(off, _CH)], isk)
                pltpu.sync_copy(dst_hbm.at[pl.ds(off, _CH)], idk)
                pltpu.async_copy(h_hbm.at[isk], rw, gk)
        return 0
    lax.fori_loop(0, _NCHUNK // 3, _iter, 0)
    for rw, isk, idk, gk, sk in slots:
        pltpu.make_async_copy(rw, acc.at[idk], sk).wait()

    ot = e0 + _NCHUNK * _CH
    pltpu.sync_copy(src_hbm.at[pl.ds(ot, _TAIL)], ist)
    pltpu.sync_copy(dst_hbm.at[pl.ds(ot, _TAIL)], idt)
    pltpu.async_copy(h_hbm.at[ist], rows0.at[pl.ds(0, _TAIL)], g0).wait()
    pltpu.sync_copy(rows0.at[pl.ds(0, _TAIL)], acc.at[idt], add=True)
    plsc.subcore_barrier()

    def _wb_in(idx):
        r0b, nb = _BLOCKS[idx]
        rw = slots[idx % 3][0]
        return (acc.at[pl.ds(s * _RPS + r0b, nb)], rw.at[pl.ds(0, nb)],
                slots[idx % 3][3])
    def _wb_out(idx):
        r0b, nb = _BLOCKS[idx]
        rw = slots[idx % 3][0]
        return (rw.at[pl.ds(0, nb)], out_hbm.at[c, pl.ds(s * _RPS + r0b, nb)],
                slots[idx % 3][4])
    for idx in range(len(_BLOCKS)):
        if idx >= 3:
            pltpu.make_async_copy(*_wb_out(idx - 3)).wait()
        pltpu.async_copy(*_wb_in(idx))
        pltpu.make_async_copy(*_wb_in(idx)).wait()
        pltpu.async_copy(*_wb_out(idx))
    for idx in range(max(0, len(_BLOCKS) - 3), len(_BLOCKS)):
        pltpu.make_async_copy(*_wb_out(idx)).wait()


def _segsum(h, src, dst):
    mesh = plsc.VectorSubcoreMesh(core_axis_name="c", subcore_axis_name="s")
    f = pl.kernel(
        _segsum_body,
        mesh=mesh,
        out_type=jax.ShapeDtypeStruct((_NC, _ROWS, _D), jnp.float32),
        scratch_types=[
            pltpu.VMEM_SHARED((_ROWS, _D), jnp.float32),
        ] + [
            t for _k in range(3) for t in (
                pltpu.VMEM((_CH, _D), jnp.float32),
                pltpu.VMEM((_CH,), jnp.int32),
                pltpu.VMEM((_CH,), jnp.int32),
                pltpu.SemaphoreType.DMA,
                pltpu.SemaphoreType.DMA,
            )
        ] + [
            pltpu.VMEM((_TAIL,), jnp.int32),
            pltpu.VMEM((_TAIL,), jnp.int32),
        ],
    )
    return f(h, src, dst)


def _matmul_t(x, w):
    return lax.dot_general(x, w, (((1,), (1,)), ((), ())),
                           preferred_element_type=jnp.float32)


def _lin_body(x_ref, w_ref, b_ref, o_ref):
    o_ref[...] = _matmul_t(x_ref[...], w_ref[...]) + b_ref[...]


def _lin(x, p):
    return pl.pallas_call(
        _lin_body,
        out_shape=jax.ShapeDtypeStruct((_N, _D), jnp.float32),
    )(x, p["W"], p["b"].reshape(1, -1))


def _mlp_core(x_ref, a_ref, eps_ref, w1_ref, b1_ref, g_ref, be_ref, w2_ref, b2_ref):
    agg = a_ref[0, 0:_N, :] + a_ref[1, 0:_N, :]
    h = (1.0 + eps_ref[0]) * x_ref[...] + agg
    t = _matmul_t(h, w1_ref[...]) + b1_ref[...]
    mean = jnp.mean(t, axis=0, keepdims=True)
    var = jnp.mean((t - mean) ** 2, axis=0, keepdims=True)
    t = (t - mean) * lax.rsqrt(var + 1e-5) * g_ref[...] + be_ref[...]
    t = jnp.maximum(t, 0.0)
    t = _matmul_t(t, w2_ref[...]) + b2_ref[...]
    return jnp.maximum(t, 0.0)


def _gin_mlp_body(x_ref, a_ref, eps_ref, w1_ref, b1_ref, g_ref, be_ref,
                  w2_ref, b2_ref, o_ref):
    o_ref[...] = _mlp_core(x_ref, a_ref, eps_ref, w1_ref, b1_ref, g_ref,
                           be_ref, w2_ref, b2_ref)


def _gin_mlp_final_body(x_ref, a_ref, eps_ref, w1_ref, b1_ref, g_ref, be_ref,
                        w2_ref, b2_ref, wf_ref, bf_ref, o_ref):
    t = _mlp_core(x_ref, a_ref, eps_ref, w1_ref, b1_ref, g_ref,
                  be_ref, w2_ref, b2_ref)
    o_ref[...] = _matmul_t(t, wf_ref[...]) + bf_ref[...]


def _mlp_args(x, agg, p):
    return (x, agg, p["eps"].reshape(1),
            p["W1"], p["b1"].reshape(1, -1),
            p["gamma"].reshape(1, -1), p["beta"].reshape(1, -1),
            p["W2"], p["b2"].reshape(1, -1))


_SMEM1 = pl.BlockSpec(memory_space=pltpu.SMEM)


def _gin_mlp(x, agg, p):
    specs = [None, None, _SMEM1] + [None] * 6
    specs = [s if s is not None else pl.BlockSpec() for s in specs]
    return pl.pallas_call(
        _gin_mlp_body,
        in_specs=specs,
        out_shape=jax.ShapeDtypeStruct((_N, _D), jnp.float32),
    )(*_mlp_args(x, agg, p))


def _gin_mlp_final(x, agg, p, pf):
    specs = [None, None, _SMEM1] + [None] * 8
    specs = [s if s is not None else pl.BlockSpec() for s in specs]
    return pl.pallas_call(
        _gin_mlp_final_body,
        in_specs=specs,
        out_shape=jax.ShapeDtypeStruct((_N, pf["W"].shape[0]), jnp.float32),
    )(*_mlp_args(x, agg, p), pf["W"], pf["b"].reshape(1, -1))


def kernel(x_author, x_paper, params, ei_writes, ei_written):
    p = params
    src_w, dst_w = ei_writes[0], ei_writes[1]
    src_n, dst_n = ei_written[0], ei_written[1]

    h_a = _lin(x_author, p["lin_author"])
    h_p = _lin(x_paper, p["lin_paper"])

    l1, l2 = p["layers"][0], p["layers"][1]
    agg_p = _segsum(h_a, src_w, dst_w)
    agg_a = _segsum(h_p, src_n, dst_n)
    h_p1 = _gin_mlp(h_p, agg_p, l1["writes"])
    h_a1 = _gin_mlp(h_a, agg_a, l1["written"])

    agg_a2 = _segsum(h_p1, src_n, dst_n)
    return _gin_mlp_final(h_a1, agg_a2, l2["written"], p["final"])
